# baseline (speedup 1.0000x reference)
import jax
import jax.numpy as jnp
from jax.experimental import pallas as pl
from jax.experimental.pallas import tpu as pltpu

_PROB = 0.05
_LO = _PROB / 2.0
_HI = 1.0 - _PROB / 2.0

_LANES = 1024
_BLK_R = 512


def _reduce_body(x_ref, mx_ref, mn_ref):
    i = pl.program_id(0)
    bmax = jnp.max(x_ref[...])
    bmin = jnp.min(x_ref[...])

    @pl.when(i == 0)
    def _init():
        mx_ref[0, 0] = bmax
        mn_ref[0, 0] = bmin

    @pl.when(i > 0)
    def _acc():
        mx_ref[0, 0] = jnp.maximum(mx_ref[0, 0], bmax)
        mn_ref[0, 0] = jnp.minimum(mn_ref[0, 0], bmin)


def _select_body(mx_ref, mn_ref, x_ref, n_ref, y_ref):
    salt = mx_ref[0, 0]
    pepper = mn_ref[0, 0]
    xb = x_ref[...]
    nb = n_ref[...]
    y = jnp.where(nb < _LO, salt, xb)
    y_ref[...] = jnp.where(nb > _HI, pepper, y)


def kernel(x, noise):
    shape = x.shape
    n = x.size
    rows = n // _LANES
    x2 = x.reshape(rows, _LANES)
    n2 = noise.reshape(rows, _LANES)
    grid = rows // _BLK_R

    mx, mn = pl.pallas_call(
        _reduce_body,
        grid=(grid,),
        in_specs=[pl.BlockSpec((_BLK_R, _LANES), lambda i: (i, 0))],
        out_specs=[
            pl.BlockSpec(memory_space=pltpu.SMEM),
            pl.BlockSpec(memory_space=pltpu.SMEM),
        ],
        out_shape=[
            jax.ShapeDtypeStruct((1, 1), jnp.float32),
            jax.ShapeDtypeStruct((1, 1), jnp.float32),
        ],
        compiler_params=pltpu.CompilerParams(
            dimension_semantics=("arbitrary",)),
    )(x2)

    y = pl.pallas_call(
        _select_body,
        grid=(grid,),
        in_specs=[
            pl.BlockSpec(memory_space=pltpu.SMEM),
            pl.BlockSpec(memory_space=pltpu.SMEM),
            pl.BlockSpec((_BLK_R, _LANES), lambda i: (i, 0)),
            pl.BlockSpec((_BLK_R, _LANES), lambda i: (i, 0)),
        ],
        out_specs=pl.BlockSpec((_BLK_R, _LANES), lambda i: (i, 0)),
        out_shape=jax.ShapeDtypeStruct((rows, _LANES), jnp.float32),
        compiler_params=pltpu.CompilerParams(
            dimension_semantics=("parallel",)),
    )(mx, mn, x2, n2)

    return y.reshape(shape)


# speedup vs baseline: 1.0286x; 1.0286x over previous
import jax
import jax.numpy as jnp
from jax.experimental import pallas as pl
from jax.experimental.pallas import tpu as pltpu

_PROB = 0.05
_LO = _PROB / 2.0
_HI = 1.0 - _PROB / 2.0

_LANES = 1024
_BLK_R = 512


def _reduce_body(x_ref, mx_ref, mn_ref, amax_ref, amin_ref):
    i = pl.program_id(0)
    xb = x_ref[...].reshape(_BLK_R // 8, 8, _LANES)
    bmax = jnp.max(xb, axis=0)
    bmin = jnp.min(xb, axis=0)

    @pl.when(i == 0)
    def _init():
        amax_ref[...] = bmax
        amin_ref[...] = bmin

    @pl.when(i > 0)
    def _acc():
        amax_ref[...] = jnp.maximum(amax_ref[...], bmax)
        amin_ref[...] = jnp.minimum(amin_ref[...], bmin)

    @pl.when(i == pl.num_programs(0) - 1)
    def _fin():
        mx_ref[0, 0] = jnp.max(amax_ref[...])
        mn_ref[0, 0] = jnp.min(amin_ref[...])


def _select_body(mx_ref, mn_ref, x_ref, n_ref, y_ref):
    salt = mx_ref[0, 0]
    pepper = mn_ref[0, 0]
    xb = x_ref[...]
    nb = n_ref[...]
    y = jnp.where(nb < _LO, salt, xb)
    y_ref[...] = jnp.where(nb > _HI, pepper, y)


def kernel(x, noise):
    shape = x.shape
    n = x.size
    rows = n // _LANES
    x2 = x.reshape(rows, _LANES)
    n2 = noise.reshape(rows, _LANES)
    grid = rows // _BLK_R

    mx, mn = pl.pallas_call(
        _reduce_body,
        grid=(grid,),
        in_specs=[pl.BlockSpec((_BLK_R, _LANES), lambda i: (i, 0))],
        out_specs=[
            pl.BlockSpec(memory_space=pltpu.SMEM),
            pl.BlockSpec(memory_space=pltpu.SMEM),
        ],
        out_shape=[
            jax.ShapeDtypeStruct((1, 1), jnp.float32),
            jax.ShapeDtypeStruct((1, 1), jnp.float32),
        ],
        scratch_shapes=[
            pltpu.VMEM((8, _LANES), jnp.float32),
            pltpu.VMEM((8, _LANES), jnp.float32),
        ],
        compiler_params=pltpu.CompilerParams(
            dimension_semantics=("arbitrary",)),
    )(x2)

    y = pl.pallas_call(
        _select_body,
        grid=(grid,),
        in_specs=[
            pl.BlockSpec(memory_space=pltpu.SMEM),
            pl.BlockSpec(memory_space=pltpu.SMEM),
            pl.BlockSpec((_BLK_R, _LANES), lambda i: (i, 0)),
            pl.BlockSpec((_BLK_R, _LANES), lambda i: (i, 0)),
        ],
        out_specs=pl.BlockSpec((_BLK_R, _LANES), lambda i: (i, 0)),
        out_shape=jax.ShapeDtypeStruct((rows, _LANES), jnp.float32),
        compiler_params=pltpu.CompilerParams(
            dimension_semantics=("parallel",)),
    )(mx, mn, x2, n2)

    return y.reshape(shape)


# speedup vs baseline: 3.3088x; 3.2169x over previous
import jax
import jax.numpy as jnp
from jax.experimental import pallas as pl
from jax.experimental.pallas import tpu as pltpu

_PROB = 0.05
_LO = _PROB / 2.0
_HI = 1.0 - _PROB / 2.0

_LANES = 512
_BLK_R = 1024


def _reduce_body(x_ref, mx_ref, mn_ref, amax_ref, amin_ref):
    i = pl.program_id(0)
    xb = x_ref[...].reshape(_BLK_R // 8, 8, _LANES)
    bmax = jnp.max(xb, axis=0)
    bmin = jnp.min(xb, axis=0)

    @pl.when(i == 0)
    def _init():
        amax_ref[...] = bmax
        amin_ref[...] = bmin

    @pl.when(i > 0)
    def _acc():
        amax_ref[...] = jnp.maximum(amax_ref[...], bmax)
        amin_ref[...] = jnp.minimum(amin_ref[...], bmin)

    @pl.when(i == pl.num_programs(0) - 1)
    def _fin():
        mx_ref[0, 0] = jnp.max(amax_ref[...])
        mn_ref[0, 0] = jnp.min(amin_ref[...])


def _select_body(mx_ref, mn_ref, x_ref, n_ref, y_ref):
    salt = mx_ref[0, 0]
    pepper = mn_ref[0, 0]
    xb = x_ref[...]
    nb = n_ref[...]
    y = jnp.where(nb < _LO, salt, xb)
    y_ref[...] = jnp.where(nb > _HI, pepper, y)


def kernel(x, noise):
    shape = x.shape
    n = x.size
    rows = n // _LANES
    x2 = x.reshape(rows, _LANES)
    n2 = noise.reshape(rows, _LANES)
    grid = rows // _BLK_R

    mx, mn = pl.pallas_call(
        _reduce_body,
        grid=(grid,),
        in_specs=[pl.BlockSpec((_BLK_R, _LANES), lambda i: (i, 0))],
        out_specs=[
            pl.BlockSpec(memory_space=pltpu.SMEM),
            pl.BlockSpec(memory_space=pltpu.SMEM),
        ],
        out_shape=[
            jax.ShapeDtypeStruct((1, 1), jnp.float32),
            jax.ShapeDtypeStruct((1, 1), jnp.float32),
        ],
        scratch_shapes=[
            pltpu.VMEM((8, _LANES), jnp.float32),
            pltpu.VMEM((8, _LANES), jnp.float32),
        ],
        compiler_params=pltpu.CompilerParams(
            dimension_semantics=("arbitrary",)),
    )(x2)

    y = pl.pallas_call(
        _select_body,
        grid=(grid,),
        in_specs=[
            pl.BlockSpec(memory_space=pltpu.SMEM),
            pl.BlockSpec(memory_space=pltpu.SMEM),
            pl.BlockSpec((_BLK_R, _LANES), lambda i: (i, 0)),
            pl.BlockSpec((_BLK_R, _LANES), lambda i: (i, 0)),
        ],
        out_specs=pl.BlockSpec((_BLK_R, _LANES), lambda i: (i, 0)),
        out_shape=jax.ShapeDtypeStruct((rows, _LANES), jnp.float32),
        compiler_params=pltpu.CompilerParams(
            dimension_semantics=("parallel",)),
    )(mx, mn, x2, n2)

    return y.reshape(shape)
